# Initial kernel scaffold; baseline (speedup 1.0000x reference)
import jax
import jax.numpy as jnp
from jax.experimental import pallas as pl


def kernel(node_input, node_attr, edge_index, edge_vec, edge_attr, batch, Wh, Uf, M1, M2, Wout):
    raise NotImplementedError("write your pallas kernel here")


# speedup vs baseline: 2.2879x; 2.2879x over previous
import functools
import math

import jax
import jax.numpy as jnp
from jax import lax
from jax.experimental import pallas as pl
from jax.experimental.pallas import tpu as pltpu
from jax.experimental.pallas import tpu_sc as plsc

F32 = jnp.float32

_NB = 10
_MIN_R = 0.0
_MAX_R = 2.0
_INV_SQRT_NEIGH = 1.0 / math.sqrt(32.0)
_INV_SQRT_POOL = 1.0 / math.sqrt(1000.0)
_NUM_GRAPHS = 10
_LAYERS = 3

_NC = 2
_NS = 16
_NW = _NC * _NS
_CHUNK = 128
_BT = 2048
_BN = 400
_D = 128


def _edge_gates_kernel(f8_ref, uf_ref, m1_ref, m2_ref, g0_ref, g1_ref, g2_ref):
    f8 = f8_ref[...]
    x = f8[0:1]
    y = f8[1:2]
    z = f8[2:3]
    n = jnp.sqrt(x * x + y * y + z * z)
    inv = 1.0 / jnp.maximum(n, 1e-12)
    ux = x * inv
    uy = y * inv
    uz = z * inv
    s3 = math.sqrt(3.0)
    s5 = math.sqrt(5.0)
    s15 = math.sqrt(15.0)
    one = jnp.ones_like(ux)
    sh = jnp.concatenate(
        [one, s3 * ux, s3 * uy, s3 * uz, s15 * ux * uy, s15 * uy * uz,
         (s5 / 2.0) * (3.0 * uz * uz - 1.0), s15 * ux * uz,
         (s15 / 2.0) * (ux * ux - uy * uy)], axis=0)
    ea13 = jnp.concatenate([f8[3:7], sh], axis=0)

    step = (_MAX_R - _MIN_R) / (_NB + 1)
    vals = ((lax.broadcasted_iota(jnp.int32, (_NB, 1), 0) + 1).astype(F32)
            * step + _MIN_R)
    diff = (n - vals) / step
    emb = (jnp.cos((math.pi / 2.0) * diff)
           * ((diff > -1.0) & (diff < 1.0)).astype(F32)
           * math.sqrt(float(_NB)))

    for l, gref in enumerate((g0_ref, g1_ref, g2_ref)):
        pre = lax.dot_general(emb, m1_ref[l], (((0,), (0,)), ((), ())),
                              preferred_element_type=F32)
        hid = pre * (1.0 / (1.0 + jnp.exp(-pre)))
        radial = jnp.dot(hid, m2_ref[l], preferred_element_type=F32)
        f = lax.dot_general(ea13, uf_ref[l], (((0,), (0,)), ((), ())),
                            preferred_element_type=F32)
        gref[...] = f * radial


def _xw_first_kernel(ni_ref, na_ref, wh_ref, x_ref, xw_ref):
    xb = ni_ref[...] * na_ref[...]
    x_ref[...] = xb
    xw_ref[...] = jnp.dot(xb, wh_ref[...], preferred_element_type=F32)


def _xw_mid_kernel(x_ref, agg_ref, wout_ref, wh_ref, xn_ref, xw_ref):
    a = (agg_ref[0] + agg_ref[1]) * _INV_SQRT_NEIGH
    xn = x_ref[...] + jnp.dot(a, wout_ref[...], preferred_element_type=F32)
    xn_ref[...] = xn
    xw_ref[...] = jnp.dot(xn, wh_ref[...], preferred_element_type=F32)


def _pool_kernel(x_ref, agg_ref, wout_ref, b_ref, out_ref):
    i = pl.program_id(0)
    a = (agg_ref[0] + agg_ref[1]) * _INV_SQRT_NEIGH
    xn = x_ref[...] + jnp.dot(a, wout_ref[...], preferred_element_type=F32)
    bcol = b_ref[...]
    onehot = (bcol == lax.broadcasted_iota(jnp.int32, (_BN, 16), 1)).astype(F32)
    ph = lax.dot_general(onehot, xn, (((0,), (0,)), ((), ())),
                         preferred_element_type=F32)

    @pl.when(i == 0)
    def _():
        out_ref[...] = jnp.zeros_like(out_ref)

    out_ref[...] = out_ref[...] + ph * _INV_SQRT_POOL


def _sc_agg_body(npad, kchunks, xw_hbm, src_hbm, dst_hbm, g_hbm, zeros_hbm,
                 out_hbm, src_v, dst_v, rows_v, g_v, agg_sh, sem):
    c = lax.axis_index("c")
    s = lax.axis_index("s")
    wid = c * _NS + s
    rps = npad // _NS

    pltpu.sync_copy(zeros_hbm.at[pl.ds(s * rps, rps)],
                    agg_sh.at[pl.ds(s * rps, rps)])
    plsc.subcore_barrier()

    def chunk(k, carry):
        base = (wid * kchunks + k) * _CHUNK
        pltpu.sync_copy(src_hbm.at[pl.ds(base, _CHUNK)], src_v)
        pltpu.async_copy(xw_hbm.at[src_v], rows_v, sem).wait()
        pltpu.sync_copy(g_hbm.at[pl.ds(base, _CHUNK)], g_v)
        pltpu.sync_copy(dst_hbm.at[pl.ds(base, _CHUNK)], dst_v)

        def row(e, cc):
            for j in range(_D // 16):
                sl = pl.ds(j * 16, 16)
                rows_v[e, sl] = rows_v[e, sl] * g_v[e, sl]
            return cc

        lax.fori_loop(0, _CHUNK, row, 0)
        pltpu.sync_copy(rows_v, agg_sh.at[dst_v], add=True)
        return carry

    lax.fori_loop(0, kchunks, chunk, 0)
    plsc.subcore_barrier()
    pltpu.sync_copy(agg_sh.at[pl.ds(s * rps, rps)],
                    out_hbm.at[pl.ds(c * npad + s * rps, rps)])


def _make_sc_agg(n_nodes, npad, kchunks):
    mesh = plsc.VectorSubcoreMesh(core_axis_name="c", subcore_axis_name="s",
                                  num_cores=_NC, num_subcores=_NS)
    return pl.kernel(
        functools.partial(_sc_agg_body, npad, kchunks),
        out_type=jax.ShapeDtypeStruct((_NC * npad, _D), F32),
        mesh=mesh,
        scratch_types=[
            pltpu.VMEM((_CHUNK,), jnp.int32),
            pltpu.VMEM((_CHUNK,), jnp.int32),
            pltpu.VMEM((_CHUNK, _D), F32),
            pltpu.VMEM((_CHUNK, _D), F32),
            pltpu.VMEM_SHARED((npad, _D), F32),
            pltpu.SemaphoreType.DMA,
        ],
    )


@jax.jit
def _impl(node_input, node_attr, edge_index, edge_vec, edge_attr, batch,
          Wh, Uf, M1, M2, Wout):
    n_nodes = node_input.shape[0]
    e = edge_index.shape[1]

    src = edge_index[1].astype(jnp.int32)
    dst = edge_index[0].astype(jnp.int32)

    grain = _NW * _CHUNK
    epad = ((e + grain - 1) // grain) * grain
    pe = epad - e
    src_p = jnp.concatenate([src, jnp.zeros((pe,), jnp.int32)])
    dst_p = jnp.concatenate([dst, jnp.full((pe,), n_nodes, jnp.int32)])

    f8 = jnp.concatenate(
        [edge_vec.T, edge_attr.T, jnp.zeros((1, e), F32)], axis=0)
    f8 = jnp.pad(f8, ((0, 0), (0, pe)))

    align = _NS * 8
    npad = ((n_nodes + 1 + align - 1) // align) * align
    kchunks = epad // (_NW * _CHUNK)

    gates = pl.pallas_call(
        _edge_gates_kernel,
        grid=(epad // _BT,),
        in_specs=[
            pl.BlockSpec((8, _BT), lambda i: (0, i)),
            pl.BlockSpec(Uf.shape, lambda i: (0, 0, 0)),
            pl.BlockSpec(M1.shape, lambda i: (0, 0, 0)),
            pl.BlockSpec(M2.shape, lambda i: (0, 0, 0)),
        ],
        out_specs=[pl.BlockSpec((_BT, _D), lambda i: (i, 0))] * _LAYERS,
        out_shape=[jax.ShapeDtypeStruct((epad, _D), F32)] * _LAYERS,
    )(f8, Uf, M1, M2)

    zeros_agg = jnp.zeros((npad, _D), F32)
    sc_agg = _make_sc_agg(n_nodes, npad, kchunks)

    x, xw = pl.pallas_call(
        _xw_first_kernel,
        grid=(n_nodes // _BN,),
        in_specs=[
            pl.BlockSpec((_BN, _D), lambda i: (i, 0)),
            pl.BlockSpec((_BN, 1), lambda i: (i, 0)),
            pl.BlockSpec((_D, _D), lambda i: (0, 0)),
        ],
        out_specs=[pl.BlockSpec((_BN, _D), lambda i: (i, 0))] * 2,
        out_shape=[jax.ShapeDtypeStruct((n_nodes, _D), F32)] * 2,
    )(node_input, node_attr, Wh[0])

    batch_col = batch.astype(jnp.int32).reshape(n_nodes, 1)

    for l in range(_LAYERS):
        agg = sc_agg(xw, src_p, dst_p, gates[l], zeros_agg)
        agg = agg.reshape(_NC, npad, _D)
        if l + 1 < _LAYERS:
            x, xw = pl.pallas_call(
                _xw_mid_kernel,
                grid=(n_nodes // _BN,),
                in_specs=[
                    pl.BlockSpec((_BN, _D), lambda i: (i, 0)),
                    pl.BlockSpec((_NC, _BN, _D), lambda i: (0, i, 0)),
                    pl.BlockSpec((_D, _D), lambda i: (0, 0)),
                    pl.BlockSpec((_D, _D), lambda i: (0, 0)),
                ],
                out_specs=[pl.BlockSpec((_BN, _D), lambda i: (i, 0))] * 2,
                out_shape=[jax.ShapeDtypeStruct((n_nodes, _D), F32)] * 2,
            )(x, agg, Wout[l], Wh[l + 1])
        else:
            out16 = pl.pallas_call(
                _pool_kernel,
                grid=(n_nodes // _BN,),
                in_specs=[
                    pl.BlockSpec((_BN, _D), lambda i: (i, 0)),
                    pl.BlockSpec((_NC, _BN, _D), lambda i: (0, i, 0)),
                    pl.BlockSpec((_D, _D), lambda i: (0, 0)),
                    pl.BlockSpec((_BN, 1), lambda i: (i, 0)),
                ],
                out_specs=pl.BlockSpec((16, _D), lambda i: (0, 0)),
                out_shape=jax.ShapeDtypeStruct((16, _D), F32),
            )(x, agg, Wout[l], batch_col)

    return out16[:_NUM_GRAPHS]


def kernel(node_input, node_attr, edge_index, edge_vec, edge_attr, batch,
           Wh, Uf, M1, M2, Wout):
    return _impl(node_input, node_attr, edge_index, edge_vec, edge_attr,
                 batch, Wh, Uf, M1, M2, Wout)


# speedup vs baseline: 2.5686x; 1.1227x over previous
import functools
import math

import jax
import jax.numpy as jnp
from jax import lax
from jax.experimental import pallas as pl
from jax.experimental.pallas import tpu as pltpu
from jax.experimental.pallas import tpu_sc as plsc

F32 = jnp.float32

_NB = 10
_MIN_R = 0.0
_MAX_R = 2.0
_INV_SQRT_NEIGH = 1.0 / math.sqrt(32.0)
_INV_SQRT_POOL = 1.0 / math.sqrt(1000.0)
_NUM_GRAPHS = 10
_LAYERS = 3

_NC = 2
_NS = 16
_NW = _NC * _NS
_CHUNK = 128
_BT = 2048
_BN = 400
_D = 128


def _edge_gates_kernel(f8_ref, uf_ref, m1_ref, m2_ref, g0_ref, g1_ref, g2_ref):
    f8 = f8_ref[...]
    x = f8[0:1]
    y = f8[1:2]
    z = f8[2:3]
    n = jnp.sqrt(x * x + y * y + z * z)
    inv = 1.0 / jnp.maximum(n, 1e-12)
    ux = x * inv
    uy = y * inv
    uz = z * inv
    s3 = math.sqrt(3.0)
    s5 = math.sqrt(5.0)
    s15 = math.sqrt(15.0)
    one = jnp.ones_like(ux)
    sh = jnp.concatenate(
        [one, s3 * ux, s3 * uy, s3 * uz, s15 * ux * uy, s15 * uy * uz,
         (s5 / 2.0) * (3.0 * uz * uz - 1.0), s15 * ux * uz,
         (s15 / 2.0) * (ux * ux - uy * uy)], axis=0)
    ea13 = jnp.concatenate([f8[3:7], sh], axis=0)

    step = (_MAX_R - _MIN_R) / (_NB + 1)
    vals = ((lax.broadcasted_iota(jnp.int32, (_NB, 1), 0) + 1).astype(F32)
            * step + _MIN_R)
    diff = (n - vals) / step
    emb = (jnp.cos((math.pi / 2.0) * diff)
           * ((diff > -1.0) & (diff < 1.0)).astype(F32)
           * math.sqrt(float(_NB)))

    for l, gref in enumerate((g0_ref, g1_ref, g2_ref)):
        pre = lax.dot_general(emb, m1_ref[l], (((0,), (0,)), ((), ())),
                              preferred_element_type=F32)
        hid = pre * (1.0 / (1.0 + jnp.exp(-pre)))
        radial = jnp.dot(hid, m2_ref[l], preferred_element_type=F32)
        f = lax.dot_general(ea13, uf_ref[l], (((0,), (0,)), ((), ())),
                            preferred_element_type=F32)
        gref[...] = f * radial


def _xw_first_kernel(ni_ref, na_ref, wh_ref, x_ref, xw_ref):
    xb = ni_ref[...] * na_ref[...]
    x_ref[...] = xb
    xw_ref[...] = jnp.dot(xb, wh_ref[...], preferred_element_type=F32)


def _xw_mid_kernel(x_ref, agg_ref, wout_ref, wh_ref, xn_ref, xw_ref):
    a = (agg_ref[0] + agg_ref[1]) * _INV_SQRT_NEIGH
    xn = x_ref[...] + jnp.dot(a, wout_ref[...], preferred_element_type=F32)
    xn_ref[...] = xn
    xw_ref[...] = jnp.dot(xn, wh_ref[...], preferred_element_type=F32)


def _pool_kernel(x_ref, agg_ref, wout_ref, b_ref, out_ref):
    i = pl.program_id(0)
    a = (agg_ref[0] + agg_ref[1]) * _INV_SQRT_NEIGH
    xn = x_ref[...] + jnp.dot(a, wout_ref[...], preferred_element_type=F32)
    bcol = b_ref[...]
    onehot = (bcol == lax.broadcasted_iota(jnp.int32, (_BN, 16), 1)).astype(F32)
    ph = lax.dot_general(onehot, xn, (((0,), (0,)), ((), ())),
                         preferred_element_type=F32)

    @pl.when(i == 0)
    def _():
        out_ref[...] = jnp.zeros_like(out_ref)

    out_ref[...] = out_ref[...] + ph * _INV_SQRT_POOL


def _sc_agg_body(npad, kchunks, xw_hbm, src_hbm, dst_hbm, g_hbm, zeros_hbm,
                 out_hbm, src_v, dst_v, rows_v, g_v, agg_sh, sem, sem2):
    c = lax.axis_index("c")
    s = lax.axis_index("s")
    wid = c * _NS + s
    rps = npad // _NS

    pltpu.sync_copy(zeros_hbm.at[pl.ds(s * rps, rps)],
                    agg_sh.at[pl.ds(s * rps, rps)])
    plsc.subcore_barrier()

    def chunk(k, carry):
        base = (wid * kchunks + k) * _CHUNK
        pltpu.sync_copy(src_hbm.at[pl.ds(base, _CHUNK)], src_v)
        d1 = pltpu.async_copy(xw_hbm.at[src_v], rows_v, sem)
        d2 = pltpu.async_copy(g_hbm.at[pl.ds(base, _CHUNK)], g_v, sem2)
        pltpu.sync_copy(dst_hbm.at[pl.ds(base, _CHUNK)], dst_v)
        d1.wait()
        d2.wait()

        @plsc.parallel_loop(0, _CHUNK, unroll=2)
        def _(e):
            for j in range(_D // 16):
                sl = pl.ds(j * 16, 16)
                rows_v[e, sl] = rows_v[e, sl] * g_v[e, sl]

        pltpu.sync_copy(rows_v, agg_sh.at[dst_v], add=True)
        return carry

    lax.fori_loop(0, kchunks, chunk, 0)
    plsc.subcore_barrier()
    pltpu.sync_copy(agg_sh.at[pl.ds(s * rps, rps)],
                    out_hbm.at[pl.ds(c * npad + s * rps, rps)])


def _make_sc_agg(n_nodes, npad, kchunks):
    mesh = plsc.VectorSubcoreMesh(core_axis_name="c", subcore_axis_name="s",
                                  num_cores=_NC, num_subcores=_NS)
    return pl.kernel(
        functools.partial(_sc_agg_body, npad, kchunks),
        out_type=jax.ShapeDtypeStruct((_NC * npad, _D), F32),
        mesh=mesh,
        scratch_types=[
            pltpu.VMEM((_CHUNK,), jnp.int32),
            pltpu.VMEM((_CHUNK,), jnp.int32),
            pltpu.VMEM((_CHUNK, _D), F32),
            pltpu.VMEM((_CHUNK, _D), F32),
            pltpu.VMEM_SHARED((npad, _D), F32),
            pltpu.SemaphoreType.DMA,
            pltpu.SemaphoreType.DMA,
        ],
    )


@jax.jit
def _impl(node_input, node_attr, edge_index, edge_vec, edge_attr, batch,
          Wh, Uf, M1, M2, Wout):
    n_nodes = node_input.shape[0]
    e = edge_index.shape[1]

    src = edge_index[1].astype(jnp.int32)
    dst = edge_index[0].astype(jnp.int32)

    grain = _NW * _CHUNK
    epad = ((e + grain - 1) // grain) * grain
    pe = epad - e
    src_p = jnp.concatenate([src, jnp.zeros((pe,), jnp.int32)])
    dst_p = jnp.concatenate([dst, jnp.full((pe,), n_nodes, jnp.int32)])

    f8 = jnp.concatenate(
        [edge_vec.T, edge_attr.T, jnp.zeros((1, e), F32)], axis=0)
    f8 = jnp.pad(f8, ((0, 0), (0, pe)))

    align = _NS * 8
    npad = ((n_nodes + 1 + align - 1) // align) * align
    kchunks = epad // (_NW * _CHUNK)
    nbn = n_nodes // _BN

    gates = pl.pallas_call(
        _edge_gates_kernel,
        grid=(epad // _BT,),
        in_specs=[
            pl.BlockSpec((8, _BT), lambda i: (0, i)),
            pl.BlockSpec(Uf.shape, lambda i: (0, 0, 0)),
            pl.BlockSpec(M1.shape, lambda i: (0, 0, 0)),
            pl.BlockSpec(M2.shape, lambda i: (0, 0, 0)),
        ],
        out_specs=[pl.BlockSpec((_BT, _D), lambda i: (i, 0))] * _LAYERS,
        out_shape=[jax.ShapeDtypeStruct((epad, _D), F32)] * _LAYERS,
    )(f8, Uf, M1, M2)

    zeros_agg = jnp.zeros((npad, _D), F32)
    sc_agg = _make_sc_agg(n_nodes, npad, kchunks)

    x, xw = pl.pallas_call(
        _xw_first_kernel,
        grid=(nbn,),
        in_specs=[
            pl.BlockSpec((_BN, _D), lambda i: (i, 0)),
            pl.BlockSpec((_BN, 1), lambda i: (i, 0)),
            pl.BlockSpec((_D, _D), lambda i: (0, 0)),
        ],
        out_specs=[pl.BlockSpec((_BN, _D), lambda i: (i, 0))] * 2,
        out_shape=[jax.ShapeDtypeStruct((n_nodes, _D), F32)] * 2,
    )(node_input, node_attr, Wh[0])

    batch_col = batch.astype(jnp.int32).reshape(n_nodes, 1)

    for l in range(_LAYERS):
        agg = sc_agg(xw, src_p, dst_p, gates[l], zeros_agg)
        agg = agg.reshape(_NC, npad, _D)
        if l + 1 < _LAYERS:
            x, xw = pl.pallas_call(
                _xw_mid_kernel,
                grid=(nbn,),
                in_specs=[
                    pl.BlockSpec((_BN, _D), lambda i: (i, 0)),
                    pl.BlockSpec((_NC, _BN, _D), lambda i: (0, i, 0)),
                    pl.BlockSpec((_D, _D), lambda i: (0, 0)),
                    pl.BlockSpec((_D, _D), lambda i: (0, 0)),
                ],
                out_specs=[pl.BlockSpec((_BN, _D), lambda i: (i, 0))] * 2,
                out_shape=[jax.ShapeDtypeStruct((n_nodes, _D), F32)] * 2,
            )(x, agg, Wout[l], Wh[l + 1])
        else:
            out16 = pl.pallas_call(
                _pool_kernel,
                grid=(nbn,),
                in_specs=[
                    pl.BlockSpec((_BN, _D), lambda i: (i, 0)),
                    pl.BlockSpec((_NC, _BN, _D), lambda i: (0, i, 0)),
                    pl.BlockSpec((_D, _D), lambda i: (0, 0)),
                    pl.BlockSpec((_BN, 1), lambda i: (i, 0)),
                ],
                out_specs=pl.BlockSpec((16, _D), lambda i: (0, 0)),
                out_shape=jax.ShapeDtypeStruct((16, _D), F32),
            )(x, agg, Wout[l], batch_col)

    return out16[:_NUM_GRAPHS]


def kernel(node_input, node_attr, edge_index, edge_vec, edge_attr, batch,
           Wh, Uf, M1, M2, Wout):
    return _impl(node_input, node_attr, edge_index, edge_vec, edge_attr,
                 batch, Wh, Uf, M1, M2, Wout)


# speedup vs baseline: 2.5858x; 1.0067x over previous
import functools
import math

import jax
import jax.numpy as jnp
from jax import lax
from jax.experimental import pallas as pl
from jax.experimental.pallas import tpu as pltpu
from jax.experimental.pallas import tpu_sc as plsc

F32 = jnp.float32

_NB = 10
_MIN_R = 0.0
_MAX_R = 2.0
_INV_SQRT_NEIGH = 1.0 / math.sqrt(32.0)
_INV_SQRT_POOL = 1.0 / math.sqrt(1000.0)
_NUM_GRAPHS = 10
_LAYERS = 3

_NC = 2
_NS = 16
_NW = _NC * _NS
_CHUNK = 128
_BT = 2048
_BN = 400
_D = 128


def _edge_gates_kernel(f8_ref, uf_ref, m1_ref, m2_ref, g0_ref, g1_ref, g2_ref):
    f8 = f8_ref[...]
    x = f8[0:1]
    y = f8[1:2]
    z = f8[2:3]
    n = jnp.sqrt(x * x + y * y + z * z)
    inv = 1.0 / jnp.maximum(n, 1e-12)
    ux = x * inv
    uy = y * inv
    uz = z * inv
    s3 = math.sqrt(3.0)
    s5 = math.sqrt(5.0)
    s15 = math.sqrt(15.0)
    one = jnp.ones_like(ux)
    sh = jnp.concatenate(
        [one, s3 * ux, s3 * uy, s3 * uz, s15 * ux * uy, s15 * uy * uz,
         (s5 / 2.0) * (3.0 * uz * uz - 1.0), s15 * ux * uz,
         (s15 / 2.0) * (ux * ux - uy * uy)], axis=0)
    ea13 = jnp.concatenate([f8[3:7], sh], axis=0)

    step = (_MAX_R - _MIN_R) / (_NB + 1)
    vals = ((lax.broadcasted_iota(jnp.int32, (_NB, 1), 0) + 1).astype(F32)
            * step + _MIN_R)
    diff = (n - vals) / step
    emb = (jnp.cos((math.pi / 2.0) * diff)
           * ((diff > -1.0) & (diff < 1.0)).astype(F32)
           * math.sqrt(float(_NB)))

    for l, gref in enumerate((g0_ref, g1_ref, g2_ref)):
        pre = lax.dot_general(emb, m1_ref[l], (((0,), (0,)), ((), ())),
                              preferred_element_type=F32)
        hid = pre * (1.0 / (1.0 + jnp.exp(-pre)))
        radial = jnp.dot(hid, m2_ref[l], preferred_element_type=F32)
        f = lax.dot_general(ea13, uf_ref[l], (((0,), (0,)), ((), ())),
                            preferred_element_type=F32)
        gref[...] = f * radial


def _xw_first_kernel(ni_ref, na_ref, wh_ref, x_ref, xw_ref):
    xb = ni_ref[...] * na_ref[...]
    x_ref[...] = xb
    xw_ref[...] = jnp.dot(xb, wh_ref[...], preferred_element_type=F32)


def _xw_mid_kernel(x_ref, agg_ref, wout_ref, wh_ref, xn_ref, xw_ref):
    a = (agg_ref[0] + agg_ref[1]) * _INV_SQRT_NEIGH
    xn = x_ref[...] + jnp.dot(a, wout_ref[...], preferred_element_type=F32)
    xn_ref[...] = xn
    xw_ref[...] = jnp.dot(xn, wh_ref[...], preferred_element_type=F32)


def _pool_kernel(x_ref, agg_ref, wout_ref, b_ref, out_ref):
    i = pl.program_id(0)
    a = (agg_ref[0] + agg_ref[1]) * _INV_SQRT_NEIGH
    xn = x_ref[...] + jnp.dot(a, wout_ref[...], preferred_element_type=F32)
    bcol = b_ref[...]
    onehot = (bcol == lax.broadcasted_iota(jnp.int32, (_BN, 16), 1)).astype(F32)
    ph = lax.dot_general(onehot, xn, (((0,), (0,)), ((), ())),
                         preferred_element_type=F32)

    @pl.when(i == 0)
    def _():
        out_ref[...] = jnp.zeros_like(out_ref)

    out_ref[...] = out_ref[...] + ph * _INV_SQRT_POOL


def _sc_agg_body(npad, kchunks, xw_hbm, src_hbm, dst_hbm, g_hbm, zeros_hbm,
                 out_hbm, sv0, sv1, dst_v, rows0, rows1, g_v, agg_sh,
                 sem0, sem1, sem2):
    c = lax.axis_index("c")
    s = lax.axis_index("s")
    wid = c * _NS + s
    rps = npad // _NS

    pltpu.sync_copy(zeros_hbm.at[pl.ds(s * rps, rps)],
                    agg_sh.at[pl.ds(s * rps, rps)])
    plsc.subcore_barrier()

    rows = (rows0, rows1)
    sv = (sv0, sv1)
    sem = (sem0, sem1)

    def base_of(k):
        return (wid * kchunks + k) * _CHUNK

    def prefetch(k, b):
        pltpu.sync_copy(src_hbm.at[pl.ds(base_of(k), _CHUNK)], sv[b])
        pltpu.async_copy(xw_hbm.at[sv[b]], rows[b], sem[b])

    def body(k, b):
        base = base_of(k)
        d2 = pltpu.async_copy(g_hbm.at[pl.ds(base, _CHUNK)], g_v, sem2)
        pltpu.sync_copy(dst_hbm.at[pl.ds(base, _CHUNK)], dst_v)
        pltpu.make_async_copy(xw_hbm.at[sv[b]], rows[b], sem[b]).wait()
        d2.wait()

        @plsc.parallel_loop(0, _CHUNK, unroll=2)
        def _(e):
            for j in range(_D // 16):
                sl = pl.ds(j * 16, 16)
                rows[b][e, sl] = rows[b][e, sl] * g_v[e, sl]

        pltpu.sync_copy(rows[b], agg_sh.at[dst_v], add=True)

    prefetch(0, 0)

    def outer(ko, carry):
        prefetch(2 * ko + 1, 1)
        body(2 * ko, 0)
        prefetch(2 * ko + 2, 0)
        body(2 * ko + 1, 1)
        return carry

    lax.fori_loop(0, kchunks // 2 - 1, outer, 0)
    prefetch(kchunks - 1, 1)
    body(kchunks - 2, 0)
    body(kchunks - 1, 1)
    plsc.subcore_barrier()
    pltpu.sync_copy(agg_sh.at[pl.ds(s * rps, rps)],
                    out_hbm.at[pl.ds(c * npad + s * rps, rps)])


def _make_sc_agg(n_nodes, npad, kchunks):
    mesh = plsc.VectorSubcoreMesh(core_axis_name="c", subcore_axis_name="s",
                                  num_cores=_NC, num_subcores=_NS)
    return pl.kernel(
        functools.partial(_sc_agg_body, npad, kchunks),
        out_type=jax.ShapeDtypeStruct((_NC * npad, _D), F32),
        mesh=mesh,
        scratch_types=[
            pltpu.VMEM((_CHUNK,), jnp.int32),
            pltpu.VMEM((_CHUNK,), jnp.int32),
            pltpu.VMEM((_CHUNK,), jnp.int32),
            pltpu.VMEM((_CHUNK, _D), F32),
            pltpu.VMEM((_CHUNK, _D), F32),
            pltpu.VMEM((_CHUNK, _D), F32),
            pltpu.VMEM_SHARED((npad, _D), F32),
            pltpu.SemaphoreType.DMA,
            pltpu.SemaphoreType.DMA,
            pltpu.SemaphoreType.DMA,
        ],
    )


@jax.jit
def _impl(node_input, node_attr, edge_index, edge_vec, edge_attr, batch,
          Wh, Uf, M1, M2, Wout):
    n_nodes = node_input.shape[0]
    e = edge_index.shape[1]

    src = edge_index[1].astype(jnp.int32)
    dst = edge_index[0].astype(jnp.int32)

    grain = _NW * _CHUNK * 2
    epad = ((e + grain - 1) // grain) * grain
    pe = epad - e
    src_p = jnp.concatenate([src, jnp.zeros((pe,), jnp.int32)])
    dst_p = jnp.concatenate([dst, jnp.full((pe,), n_nodes, jnp.int32)])

    f8 = jnp.concatenate(
        [edge_vec.T, edge_attr.T, jnp.zeros((1, e), F32)], axis=0)
    f8 = jnp.pad(f8, ((0, 0), (0, pe)))

    align = _NS * 8
    npad = ((n_nodes + 1 + align - 1) // align) * align
    kchunks = epad // (_NW * _CHUNK)
    nbn = n_nodes // _BN

    gates = pl.pallas_call(
        _edge_gates_kernel,
        grid=(epad // _BT,),
        in_specs=[
            pl.BlockSpec((8, _BT), lambda i: (0, i)),
            pl.BlockSpec(Uf.shape, lambda i: (0, 0, 0)),
            pl.BlockSpec(M1.shape, lambda i: (0, 0, 0)),
            pl.BlockSpec(M2.shape, lambda i: (0, 0, 0)),
        ],
        out_specs=[pl.BlockSpec((_BT, _D), lambda i: (i, 0))] * _LAYERS,
        out_shape=[jax.ShapeDtypeStruct((epad, _D), F32)] * _LAYERS,
    )(f8, Uf, M1, M2)

    zeros_agg = jnp.zeros((npad, _D), F32)
    sc_agg = _make_sc_agg(n_nodes, npad, kchunks)

    x, xw = pl.pallas_call(
        _xw_first_kernel,
        grid=(nbn,),
        in_specs=[
            pl.BlockSpec((_BN, _D), lambda i: (i, 0)),
            pl.BlockSpec((_BN, 1), lambda i: (i, 0)),
            pl.BlockSpec((_D, _D), lambda i: (0, 0)),
        ],
        out_specs=[pl.BlockSpec((_BN, _D), lambda i: (i, 0))] * 2,
        out_shape=[jax.ShapeDtypeStruct((n_nodes, _D), F32)] * 2,
    )(node_input, node_attr, Wh[0])

    batch_col = batch.astype(jnp.int32).reshape(n_nodes, 1)

    for l in range(_LAYERS):
        agg = sc_agg(xw, src_p, dst_p, gates[l], zeros_agg)
        agg = agg.reshape(_NC, npad, _D)
        if l + 1 < _LAYERS:
            x, xw = pl.pallas_call(
                _xw_mid_kernel,
                grid=(nbn,),
                in_specs=[
                    pl.BlockSpec((_BN, _D), lambda i: (i, 0)),
                    pl.BlockSpec((_NC, _BN, _D), lambda i: (0, i, 0)),
                    pl.BlockSpec((_D, _D), lambda i: (0, 0)),
                    pl.BlockSpec((_D, _D), lambda i: (0, 0)),
                ],
                out_specs=[pl.BlockSpec((_BN, _D), lambda i: (i, 0))] * 2,
                out_shape=[jax.ShapeDtypeStruct((n_nodes, _D), F32)] * 2,
            )(x, agg, Wout[l], Wh[l + 1])
        else:
            out16 = pl.pallas_call(
                _pool_kernel,
                grid=(nbn,),
                in_specs=[
                    pl.BlockSpec((_BN, _D), lambda i: (i, 0)),
                    pl.BlockSpec((_NC, _BN, _D), lambda i: (0, i, 0)),
                    pl.BlockSpec((_D, _D), lambda i: (0, 0)),
                    pl.BlockSpec((_BN, 1), lambda i: (i, 0)),
                ],
                out_specs=pl.BlockSpec((16, _D), lambda i: (0, 0)),
                out_shape=jax.ShapeDtypeStruct((16, _D), F32),
            )(x, agg, Wout[l], batch_col)

    return out16[:_NUM_GRAPHS]


def kernel(node_input, node_attr, edge_index, edge_vec, edge_attr, batch,
           Wh, Uf, M1, M2, Wout):
    return _impl(node_input, node_attr, edge_index, edge_vec, edge_attr,
                 batch, Wh, Uf, M1, M2, Wout)


# speedup vs baseline: 2.6858x; 1.0387x over previous
import functools
import math

import jax
import jax.numpy as jnp
from jax import lax
from jax.experimental import pallas as pl
from jax.experimental.pallas import tpu as pltpu
from jax.experimental.pallas import tpu_sc as plsc

F32 = jnp.float32

_NB = 10
_MIN_R = 0.0
_MAX_R = 2.0
_INV_SQRT_NEIGH = 1.0 / math.sqrt(32.0)
_INV_SQRT_POOL = 1.0 / math.sqrt(1000.0)
_NUM_GRAPHS = 10
_LAYERS = 3

_NC = 2
_NS = 16
_NW = _NC * _NS
_CHUNK = 128
_BT = 2048
_BN = 400
_D = 128


def _edge_gates_kernel(f8_ref, uf_ref, m1_ref, m2_ref, g0_ref, g1_ref, g2_ref):
    f8 = f8_ref[...]
    x = f8[0:1]
    y = f8[1:2]
    z = f8[2:3]
    n = jnp.sqrt(x * x + y * y + z * z)
    inv = 1.0 / jnp.maximum(n, 1e-12)
    ux = x * inv
    uy = y * inv
    uz = z * inv
    s3 = math.sqrt(3.0)
    s5 = math.sqrt(5.0)
    s15 = math.sqrt(15.0)
    one = jnp.ones_like(ux)
    sh = jnp.concatenate(
        [one, s3 * ux, s3 * uy, s3 * uz, s15 * ux * uy, s15 * uy * uz,
         (s5 / 2.0) * (3.0 * uz * uz - 1.0), s15 * ux * uz,
         (s15 / 2.0) * (ux * ux - uy * uy)], axis=0)
    ea13 = jnp.concatenate([f8[3:7], sh], axis=0)

    step = (_MAX_R - _MIN_R) / (_NB + 1)
    vals = ((lax.broadcasted_iota(jnp.int32, (_NB, 1), 0) + 1).astype(F32)
            * step + _MIN_R)
    diff = (n - vals) / step
    emb = (jnp.cos((math.pi / 2.0) * diff)
           * ((diff > -1.0) & (diff < 1.0)).astype(F32)
           * math.sqrt(float(_NB)))

    for l, gref in enumerate((g0_ref, g1_ref, g2_ref)):
        pre = lax.dot_general(emb, m1_ref[l], (((0,), (0,)), ((), ())),
                              preferred_element_type=F32)
        hid = pre * (1.0 / (1.0 + jnp.exp(-pre)))
        radial = jnp.dot(hid, m2_ref[l], preferred_element_type=F32)
        f = lax.dot_general(ea13, uf_ref[l], (((0,), (0,)), ((), ())),
                            preferred_element_type=F32)
        gref[...] = f * radial


def _xw_first_kernel(ni_ref, na_ref, wh_ref, x_ref, xw_ref):
    xb = ni_ref[...] * na_ref[...]
    x_ref[...] = xb
    xw_ref[...] = jnp.dot(xb, wh_ref[...], preferred_element_type=F32)


def _xw_mid_kernel(x_ref, agg_ref, wout_ref, wh_ref, xn_ref, xw_ref):
    a = (agg_ref[0] + agg_ref[1]) * _INV_SQRT_NEIGH
    xn = x_ref[...] + jnp.dot(a, wout_ref[...], preferred_element_type=F32)
    xn_ref[...] = xn
    xw_ref[...] = jnp.dot(xn, wh_ref[...], preferred_element_type=F32)


def _pool_kernel(x_ref, agg_ref, wout_ref, b_ref, out_ref):
    i = pl.program_id(0)
    a = (agg_ref[0] + agg_ref[1]) * _INV_SQRT_NEIGH
    xn = x_ref[...] + jnp.dot(a, wout_ref[...], preferred_element_type=F32)
    bcol = b_ref[...]
    onehot = (bcol == lax.broadcasted_iota(jnp.int32, (_BN, 16), 1)).astype(F32)
    ph = lax.dot_general(onehot, xn, (((0,), (0,)), ((), ())),
                         preferred_element_type=F32)

    @pl.when(i == 0)
    def _():
        out_ref[...] = jnp.zeros_like(out_ref)

    out_ref[...] = out_ref[...] + ph * _INV_SQRT_POOL


def _sc_agg_body(npad, kchunks, xw_hbm, src_hbm, dst_hbm, g_hbm, zeros_hbm,
                 out_hbm, sv0, sv1, dv0, dv1, rows0, rows1, g_v, agg_sh,
                 sem0, sem1, isem0, isem1, dsem0, dsem1, sem2):
    c = lax.axis_index("c")
    s = lax.axis_index("s")
    wid = c * _NS + s
    rps = npad // _NS

    pltpu.sync_copy(zeros_hbm.at[pl.ds(s * rps, rps)],
                    agg_sh.at[pl.ds(s * rps, rps)])
    plsc.subcore_barrier()

    rows = (rows0, rows1)
    sv = (sv0, sv1)
    dv = (dv0, dv1)
    sem = (sem0, sem1)
    isem = (isem0, isem1)
    dsem = (dsem0, dsem1)

    def base_of(k):
        return (wid * kchunks + k) * _CHUNK

    def body(k, b, has1, has2):
        d2 = pltpu.async_copy(g_hbm.at[pl.ds(base_of(k), _CHUNK)], g_v, sem2)
        if has1:
            pltpu.make_async_copy(src_hbm.at[pl.ds(base_of(k + 1), _CHUNK)],
                                  sv[1 - b], isem[1 - b]).wait()
            pltpu.async_copy(xw_hbm.at[sv[1 - b]], rows[1 - b], sem[1 - b])
        pltpu.make_async_copy(xw_hbm.at[sv[b]], rows[b], sem[b]).wait()
        if has2:
            pltpu.async_copy(src_hbm.at[pl.ds(base_of(k + 2), _CHUNK)],
                             sv[b], isem[b])
        pltpu.make_async_copy(dst_hbm.at[pl.ds(base_of(k), _CHUNK)],
                              dv[b], dsem[b]).wait()
        if has1:
            pltpu.async_copy(dst_hbm.at[pl.ds(base_of(k + 1), _CHUNK)],
                             dv[1 - b], dsem[1 - b])
        d2.wait()

        @plsc.parallel_loop(0, _CHUNK, unroll=2)
        def _(e):
            for j in range(_D // 16):
                sl = pl.ds(j * 16, 16)
                rows[b][e, sl] = rows[b][e, sl] * g_v[e, sl]

        pltpu.sync_copy(rows[b], agg_sh.at[dv[b]], add=True)

    pltpu.sync_copy(src_hbm.at[pl.ds(base_of(0), _CHUNK)], sv[0])
    pltpu.async_copy(xw_hbm.at[sv[0]], rows[0], sem[0])
    pltpu.async_copy(src_hbm.at[pl.ds(base_of(1), _CHUNK)], sv[1], isem[1])
    pltpu.async_copy(dst_hbm.at[pl.ds(base_of(0), _CHUNK)], dv[0], dsem[0])

    def outer(ko, carry):
        body(2 * ko, 0, True, True)
        body(2 * ko + 1, 1, True, True)
        return carry

    lax.fori_loop(0, kchunks // 2 - 1, outer, 0)
    body(kchunks - 2, 0, True, False)
    body(kchunks - 1, 1, False, False)
    plsc.subcore_barrier()
    pltpu.sync_copy(agg_sh.at[pl.ds(s * rps, rps)],
                    out_hbm.at[pl.ds(c * npad + s * rps, rps)])


def _make_sc_agg(n_nodes, npad, kchunks):
    mesh = plsc.VectorSubcoreMesh(core_axis_name="c", subcore_axis_name="s",
                                  num_cores=_NC, num_subcores=_NS)
    return pl.kernel(
        functools.partial(_sc_agg_body, npad, kchunks),
        out_type=jax.ShapeDtypeStruct((_NC * npad, _D), F32),
        mesh=mesh,
        scratch_types=[
            pltpu.VMEM((_CHUNK,), jnp.int32),
            pltpu.VMEM((_CHUNK,), jnp.int32),
            pltpu.VMEM((_CHUNK,), jnp.int32),
            pltpu.VMEM((_CHUNK,), jnp.int32),
            pltpu.VMEM((_CHUNK, _D), F32),
            pltpu.VMEM((_CHUNK, _D), F32),
            pltpu.VMEM((_CHUNK, _D), F32),
            pltpu.VMEM_SHARED((npad, _D), F32),
            pltpu.SemaphoreType.DMA,
            pltpu.SemaphoreType.DMA,
            pltpu.SemaphoreType.DMA,
            pltpu.SemaphoreType.DMA,
            pltpu.SemaphoreType.DMA,
            pltpu.SemaphoreType.DMA,
            pltpu.SemaphoreType.DMA,
        ],
    )


@jax.jit
def _impl(node_input, node_attr, edge_index, edge_vec, edge_attr, batch,
          Wh, Uf, M1, M2, Wout):
    n_nodes = node_input.shape[0]
    e = edge_index.shape[1]

    src = edge_index[1].astype(jnp.int32)
    dst = edge_index[0].astype(jnp.int32)

    grain = _NW * _CHUNK * 2
    epad = ((e + grain - 1) // grain) * grain
    pe = epad - e
    src_p = jnp.concatenate([src, jnp.zeros((pe,), jnp.int32)])
    dst_p = jnp.concatenate([dst, jnp.full((pe,), n_nodes, jnp.int32)])

    f8 = jnp.concatenate(
        [edge_vec.T, edge_attr.T, jnp.zeros((1, e), F32)], axis=0)
    f8 = jnp.pad(f8, ((0, 0), (0, pe)))

    align = _NS * 8
    npad = ((n_nodes + 1 + align - 1) // align) * align
    kchunks = epad // (_NW * _CHUNK)
    nbn = n_nodes // _BN

    gates = pl.pallas_call(
        _edge_gates_kernel,
        grid=(epad // _BT,),
        in_specs=[
            pl.BlockSpec((8, _BT), lambda i: (0, i)),
            pl.BlockSpec(Uf.shape, lambda i: (0, 0, 0)),
            pl.BlockSpec(M1.shape, lambda i: (0, 0, 0)),
            pl.BlockSpec(M2.shape, lambda i: (0, 0, 0)),
        ],
        out_specs=[pl.BlockSpec((_BT, _D), lambda i: (i, 0))] * _LAYERS,
        out_shape=[jax.ShapeDtypeStruct((epad, _D), F32)] * _LAYERS,
    )(f8, Uf, M1, M2)

    zeros_agg = jnp.zeros((npad, _D), F32)
    sc_agg = _make_sc_agg(n_nodes, npad, kchunks)

    x, xw = pl.pallas_call(
        _xw_first_kernel,
        grid=(nbn,),
        in_specs=[
            pl.BlockSpec((_BN, _D), lambda i: (i, 0)),
            pl.BlockSpec((_BN, 1), lambda i: (i, 0)),
            pl.BlockSpec((_D, _D), lambda i: (0, 0)),
        ],
        out_specs=[pl.BlockSpec((_BN, _D), lambda i: (i, 0))] * 2,
        out_shape=[jax.ShapeDtypeStruct((n_nodes, _D), F32)] * 2,
    )(node_input, node_attr, Wh[0])

    batch_col = batch.astype(jnp.int32).reshape(n_nodes, 1)

    for l in range(_LAYERS):
        agg = sc_agg(xw, src_p, dst_p, gates[l], zeros_agg)
        agg = agg.reshape(_NC, npad, _D)
        if l + 1 < _LAYERS:
            x, xw = pl.pallas_call(
                _xw_mid_kernel,
                grid=(nbn,),
                in_specs=[
                    pl.BlockSpec((_BN, _D), lambda i: (i, 0)),
                    pl.BlockSpec((_NC, _BN, _D), lambda i: (0, i, 0)),
                    pl.BlockSpec((_D, _D), lambda i: (0, 0)),
                    pl.BlockSpec((_D, _D), lambda i: (0, 0)),
                ],
                out_specs=[pl.BlockSpec((_BN, _D), lambda i: (i, 0))] * 2,
                out_shape=[jax.ShapeDtypeStruct((n_nodes, _D), F32)] * 2,
            )(x, agg, Wout[l], Wh[l + 1])
        else:
            out16 = pl.pallas_call(
                _pool_kernel,
                grid=(nbn,),
                in_specs=[
                    pl.BlockSpec((_BN, _D), lambda i: (i, 0)),
                    pl.BlockSpec((_NC, _BN, _D), lambda i: (0, i, 0)),
                    pl.BlockSpec((_D, _D), lambda i: (0, 0)),
                    pl.BlockSpec((_BN, 1), lambda i: (i, 0)),
                ],
                out_specs=pl.BlockSpec((16, _D), lambda i: (0, 0)),
                out_shape=jax.ShapeDtypeStruct((16, _D), F32),
            )(x, agg, Wout[l], batch_col)

    return out16[:_NUM_GRAPHS]


def kernel(node_input, node_attr, edge_index, edge_vec, edge_attr, batch,
           Wh, Uf, M1, M2, Wout):
    return _impl(node_input, node_attr, edge_index, edge_vec, edge_attr,
                 batch, Wh, Uf, M1, M2, Wout)
